# Initial kernel scaffold; baseline (speedup 1.0000x reference)
import jax
import jax.numpy as jnp
from jax.experimental import pallas as pl


def kernel(x, token_table, pos_table):
    raise NotImplementedError("write your pallas kernel here")


# speedup vs baseline: 4.0844x; 4.0844x over previous
import functools

import jax
import jax.numpy as jnp
from jax import lax
from jax.experimental import pallas as pl
from jax.experimental.pallas import tpu as pltpu
from jax.experimental.pallas import tpu_sc as plsc

VOCAB = 100000
MAXLEN = 200
EMBED = 128
BATCH = 4096

NUM_CORES = 2
NUM_SUBCORES = 16
NW = NUM_CORES * NUM_SUBCORES
ROWS = BATCH * MAXLEN
ROWS_PER_W = ROWS // NW
CHUNKS_PER_W = ROWS_PER_W // MAXLEN
LANES = 16
D_VECS = EMBED // LANES

_mesh = plsc.VectorSubcoreMesh(core_axis_name="c", subcore_axis_name="s")


@functools.partial(
    pl.kernel,
    mesh=_mesh,
    out_type=jax.ShapeDtypeStruct((ROWS, EMBED), jnp.float32),
    scratch_types=[
        pltpu.VMEM((ROWS_PER_W,), jnp.int32),
        pltpu.VMEM((MAXLEN, EMBED), jnp.float32),
        pltpu.VMEM((MAXLEN, EMBED), jnp.float32),
        pltpu.SemaphoreType.DMA,
    ],
)
def _emb_kernel(x_hbm, tok_hbm, pos_hbm, out_hbm, idx_v, pos_v, rows_v, sem):
    wid = lax.axis_index("s") * NUM_CORES + lax.axis_index("c")
    base = pl.multiple_of(wid * ROWS_PER_W, 8)

    pltpu.sync_copy(pos_hbm, pos_v)
    pltpu.sync_copy(x_hbm.at[pl.ds(base, ROWS_PER_W)], idx_v)

    def chunk_body(g, carry):
        off = pl.multiple_of(g * MAXLEN, 8)
        pltpu.async_copy(
            tok_hbm.at[idx_v.at[pl.ds(off, 128)]],
            rows_v.at[pl.ds(0, 128)],
            sem,
        ).wait()
        pltpu.async_copy(
            tok_hbm.at[idx_v.at[pl.ds(off + 128, 72)]],
            rows_v.at[pl.ds(128, 72)],
            sem,
        ).wait()

        def add_body(t, carry2):
            for d in range(D_VECS):
                sl = pl.ds(d * LANES, LANES)
                rows_v[t, sl] = rows_v[t, sl] + pos_v[t, sl]
            return carry2

        lax.fori_loop(0, MAXLEN, add_body, 0)

        pltpu.sync_copy(rows_v, out_hbm.at[pl.ds(base + off, MAXLEN)])
        return carry

    lax.fori_loop(0, CHUNKS_PER_W, chunk_body, 0)


def kernel(x, token_table, pos_table):
    x_flat = x.reshape(-1).astype(jnp.int32)
    out = _emb_kernel(x_flat, token_table, pos_table)
    return out.reshape(BATCH, MAXLEN, EMBED)
